# baseline (speedup 1.0000x reference)
import functools

import jax
import jax.numpy as jnp
from jax import lax
from jax.experimental import pallas as pl
from jax.experimental.pallas import tpu as pltpu
from jax.experimental.pallas import tpu_sc as plsc


def _tc_relayout(tt, D):
    V = tt.shape[1]
    BLK = 1024
    reps = 128 // D

    def body(in_ref, out_ref):
        t = jnp.swapaxes(in_ref[...], 0, 1)
        out_ref[...] = jnp.concatenate([t] * reps, axis=1)

    return pl.pallas_call(
        body,
        grid=(pl.cdiv(V, BLK),),
        in_specs=[pl.BlockSpec((D, BLK), lambda i: (0, i))],
        out_specs=pl.BlockSpec((BLK, 128), lambda i: (i, 0)),
        out_shape=jax.ShapeDtypeStruct((V, 128), jnp.float32),
    )(tt)


def _build_encoder(B, L, D, table_rows):
    info = plsc.get_sparse_core_info()
    NC, NS, LANES = info.num_cores, info.num_subcores, info.num_lanes
    NW = NC * NS
    assert B % NW == 0
    b_per_w = B // NW
    CB = 8
    assert b_per_w % CB == 0
    nchunks = b_per_w // CB
    CI = CB * L
    LP = (L + LANES - 1) // LANES * LANES
    NIV = LP // LANES
    ND = D // LANES
    assert D == ND * LANES
    XW = 128

    mesh = plsc.VectorSubcoreMesh(core_axis_name="c", subcore_axis_name="s")

    @functools.partial(
        pl.kernel,
        mesh=mesh,
        out_type=jax.ShapeDtypeStruct((B, D), jnp.float32),
        compiler_params=pltpu.CompilerParams(
            use_tc_tiling_on_sc=False, needs_layout_passes=False
        ),
        scratch_types=[
            pltpu.VMEM((b_per_w * XW,), jnp.float32),
            pltpu.VMEM((b_per_w * LP,), jnp.int32),
            pltpu.VMEM((CI, D), jnp.float32),
            pltpu.VMEM((CI, D), jnp.float32),
            pltpu.VMEM((CB, D), jnp.float32),
            pltpu.VMEM((CB, D), jnp.float32),
            pltpu.SemaphoreType.DMA,
            pltpu.SemaphoreType.DMA,
            pltpu.SemaphoreType.DMA,
            pltpu.SemaphoreType.DMA,
        ],
    )
    def encode(table_h, x_h, out_h, x_v, idx_v, rows0, rows1, outv0, outv1,
               sem0, sem1, osem0, osem1):
        wid = lax.axis_index("s") * NC + lax.axis_index("c")
        bufs = (rows0, rows1)
        sems = (sem0, sem1)
        outvs = (outv0, outv1)
        osems = (osem0, osem1)

        pltpu.sync_copy(x_h.at[pl.ds(wid * b_per_w * XW, b_per_w * XW)], x_v)

        def conv(r, carry):
            for k in range(NIV):
                v = x_v[pl.ds(r * XW + k * LANES, LANES)].astype(jnp.int32)
                idx_v[pl.ds(r * LP + k * LANES, LANES)] = v + v
            return carry

        lax.fori_loop(0, b_per_w, conv, 0)

        def fire(c):
            buf, sem = bufs[c % 2], sems[c % 2]
            for i in range(CB):
                pltpu.async_copy(
                    table_h.at[idx_v.at[pl.ds((c * CB + i) * LP, L)]],
                    buf.at[pl.ds(i * L, L)],
                    sem,
                )

        def drain(c):
            buf, sem = bufs[c % 2], sems[c % 2]
            for i in range(CB):
                pltpu.make_async_copy(
                    table_h.at[idx_v.at[pl.ds((c * CB + i) * LP, L)]],
                    buf.at[pl.ds(i * L, L)],
                    sem,
                ).wait()

        fire(0)
        for c in range(nchunks):
            if c + 1 < nchunks:
                fire(c + 1)
            drain(c)
            rows_v = bufs[c % 2]
            out_v = outvs[c % 2]
            osem = osems[c % 2]
            if c >= 2:
                pltpu.make_async_copy(
                    out_v,
                    out_h.at[pl.ds(wid * b_per_w + (c - 2) * CB, CB)],
                    osem,
                ).wait()

            def row(i, carry, c=c, rows_v=rows_v, out_v=out_v):
                accs = [jnp.zeros((LANES,), jnp.float32) for _ in range(ND)]
                b = c * CB + i
                wvecs = [
                    x_v[pl.ds(b * XW + L + k * LANES, LANES)]
                    for k in range(NIV - 1)
                ]
                wtail = x_v[pl.ds(b * XW + 2 * L - LANES, LANES)]
                for l in range(L):
                    r = i * L + l
                    if l < (NIV - 1) * LANES:
                        wl = wvecs[l // LANES][l % LANES]
                    else:
                        wl = wtail[L + l - (2 * L - LANES)]
                    for d in range(ND):
                        accs[d] = accs[d] + wl * rows_v[r, pl.ds(d * LANES, LANES)]
                for d in range(ND):
                    out_v[i, pl.ds(d * LANES, LANES)] = accs[d]
                return carry

            lax.fori_loop(0, CB, row, 0)
            pltpu.async_copy(
                out_v, out_h.at[pl.ds(wid * b_per_w + c * CB, CB)], osem
            )

        for c in (nchunks - 2, nchunks - 1):
            pltpu.make_async_copy(
                outvs[c % 2],
                out_h.at[pl.ds(wid * b_per_w + c * CB, CB)],
                osems[c % 2],
            ).wait()

    return encode


def kernel(x, table):
    B, two_l = x.shape
    L = two_l // 2
    V, D = table.shape
    enc = _build_encoder(B, L, D, V)
    xp = jnp.pad(x, ((0, 0), (0, 128 - two_l)))
    tp = _tc_relayout(jnp.swapaxes(table, 0, 1), D).reshape(2 * V, D)
    return enc(tp, xp.reshape(-1))


# speedup vs baseline: 1.2200x; 1.2200x over previous
import functools

import jax
import jax.numpy as jnp
from jax import lax
from jax.experimental import pallas as pl
from jax.experimental.pallas import tpu as pltpu
from jax.experimental.pallas import tpu_sc as plsc


def _build_encoder(B, L, D, table_rows):
    info = plsc.get_sparse_core_info()
    NC, NS, LANES = info.num_cores, info.num_subcores, info.num_lanes
    NW = NC * NS
    assert B % NW == 0
    b_per_w = B // NW
    CB = 8
    assert b_per_w % CB == 0
    nchunks = b_per_w // CB
    CI = CB * L
    LP = (L + LANES - 1) // LANES * LANES
    NIV = LP // LANES
    ND = D // LANES
    assert D == ND * LANES
    XW = 128

    mesh = plsc.VectorSubcoreMesh(core_axis_name="c", subcore_axis_name="s")

    @functools.partial(
        pl.kernel,
        mesh=mesh,
        out_type=jax.ShapeDtypeStruct((B, D), jnp.float32),
        compiler_params=pltpu.CompilerParams(
            use_tc_tiling_on_sc=False, needs_layout_passes=False
        ),
        scratch_types=[
            pltpu.VMEM((b_per_w * XW,), jnp.float32),
            pltpu.VMEM((b_per_w * LP,), jnp.int32),
            pltpu.VMEM((CI, D), jnp.float32),
            pltpu.VMEM((CI, D), jnp.float32),
            pltpu.VMEM((CB, D), jnp.float32),
            pltpu.VMEM((CB, D), jnp.float32),
            pltpu.SemaphoreType.DMA,
            pltpu.SemaphoreType.DMA,
            pltpu.SemaphoreType.DMA,
            pltpu.SemaphoreType.DMA,
        ],
    )
    def encode(table_h, x_h, out_h, x_v, idx_v, rows0, rows1, outv0, outv1,
               sem0, sem1, osem0, osem1):
        wid = lax.axis_index("s") * NC + lax.axis_index("c")
        bufs = (rows0, rows1)
        sems = (sem0, sem1)
        outvs = (outv0, outv1)
        osems = (osem0, osem1)

        pltpu.sync_copy(x_h.at[pl.ds(wid * b_per_w * XW, b_per_w * XW)], x_v)

        def conv(r, carry):
            for k in range(NIV):
                v = x_v[pl.ds(r * XW + k * LANES, LANES)].astype(jnp.int32)
                idx_v[pl.ds(r * LP + k * LANES, LANES)] = v + v
            return carry

        lax.fori_loop(0, b_per_w, conv, 0)

        def fire(c):
            buf, sem = bufs[c % 2], sems[c % 2]
            for i in range(CB):
                pltpu.async_copy(
                    table_h.at[idx_v.at[pl.ds((c * CB + i) * LP, L)]],
                    buf.at[pl.ds(i * L, L)],
                    sem,
                )

        def drain(c):
            buf, sem = bufs[c % 2], sems[c % 2]
            for i in range(CB):
                pltpu.make_async_copy(
                    table_h.at[idx_v.at[pl.ds((c * CB + i) * LP, L)]],
                    buf.at[pl.ds(i * L, L)],
                    sem,
                ).wait()

        fire(0)
        for c in range(nchunks):
            if c + 1 < nchunks:
                fire(c + 1)
            drain(c)
            rows_v = bufs[c % 2]
            out_v = outvs[c % 2]
            osem = osems[c % 2]
            if c >= 2:
                pltpu.make_async_copy(
                    out_v,
                    out_h.at[pl.ds(wid * b_per_w + (c - 2) * CB, CB)],
                    osem,
                ).wait()

            def row(i, carry, c=c, rows_v=rows_v, out_v=out_v):
                accs = [jnp.zeros((LANES,), jnp.float32) for _ in range(ND)]
                b = c * CB + i
                wvecs = [
                    x_v[pl.ds(b * XW + L + k * LANES, LANES)]
                    for k in range(NIV - 1)
                ]
                wtail = x_v[pl.ds(b * XW + 2 * L - LANES, LANES)]
                for l in range(L):
                    r = i * L + l
                    if l < (NIV - 1) * LANES:
                        wl = wvecs[l // LANES][l % LANES]
                    else:
                        wl = wtail[L + l - (2 * L - LANES)]
                    for d in range(ND):
                        accs[d] = accs[d] + wl * rows_v[r, pl.ds(d * LANES, LANES)]
                for d in range(ND):
                    out_v[i, pl.ds(d * LANES, LANES)] = accs[d]
                return carry

            lax.fori_loop(0, CB, row, 0)
            pltpu.async_copy(
                out_v, out_h.at[pl.ds(wid * b_per_w + c * CB, CB)], osem
            )

        for c in (nchunks - 2, nchunks - 1):
            pltpu.make_async_copy(
                outvs[c % 2],
                out_h.at[pl.ds(wid * b_per_w + c * CB, CB)],
                osems[c % 2],
            ).wait()

    return encode


def kernel(x, table):
    B, two_l = x.shape
    L = two_l // 2
    V, D = table.shape
    enc = _build_encoder(B, L, D, V)
    xp = jnp.pad(x, ((0, 0), (0, 128 - two_l)))
    tp = jnp.pad(table, ((0, 0), (0, 128 - D))).reshape(2 * V, D)
    return enc(tp, xp.reshape(-1))
